# baseline (speedup 1.0000x reference)
import functools

import jax
import jax.numpy as jnp
from jax import lax
from jax.experimental import pallas as pl
from jax.experimental.pallas import tpu as pltpu
from jax.experimental.pallas import tpu_sc as plsc

B_G, N_G, F_IN = 10, 1000, 128
E_EDGES = 320000
H1_DIM, H2_DIM, OUT_DIM = 64, 32, 16
NODES = B_G * N_G

NC, NS = 2, 16
CHUNK = 128
NCHUNK = 160
E_PAD = NS * NCHUNK * CHUNK
ACC_ROWS = 10240
RPT = ACC_ROWS // NS

K_GRP = 4
NGRP = NCHUNK // K_GRP


def _make_segsum(d: int):
    dh = d // 2
    mesh = plsc.VectorSubcoreMesh(core_axis_name="c", subcore_axis_name="s")

    @functools.partial(
        pl.kernel,
        out_type=pltpu.HBM((NC * ACC_ROWS, dh), jnp.float32),
        mesh=mesh,
        compiler_params=pltpu.CompilerParams(use_tc_tiling_on_sc=False),
        scratch_types=[
            pltpu.VMEM((NCHUNK, CHUNK), jnp.int32),
            pltpu.VMEM((NCHUNK, CHUNK), jnp.int32),
            pltpu.VMEM((2, K_GRP, CHUNK, dh), jnp.float32),
            pltpu.VMEM_SHARED((ACC_ROWS, dh), jnp.float32),
            pltpu.VMEM_SHARED((ACC_ROWS, dh), jnp.float32),
            pltpu.SemaphoreType.DMA((2,)),
            pltpu.SemaphoreType.DMA((2,)),
        ],
    )
    def seg(src_hbm, dst_hbm, p0_hbm, p1_hbm, zero_hbm, out_hbm,
            src_v, dst_v, rows_v, acc, ptab, gsem, ssem):
        cid = lax.axis_index("c")
        sid = lax.axis_index("s")
        pltpu.sync_copy(src_hbm.at[sid], src_v)
        pltpu.sync_copy(dst_hbm.at[sid], dst_v)
        @pl.when(cid == 0)
        def _():
            pltpu.sync_copy(p0_hbm.at[pl.ds(sid * RPT, RPT)],
                            ptab.at[pl.ds(sid * RPT, RPT)])

        @pl.when(cid == 1)
        def _():
            pltpu.sync_copy(p1_hbm.at[pl.ds(sid * RPT, RPT)],
                            ptab.at[pl.ds(sid * RPT, RPT)])

        pltpu.sync_copy(zero_hbm, acc.at[pl.ds(sid * RPT, RPT)])
        plsc.subcore_barrier()

        def gathers(h, g):
            for b in range(K_GRP):
                pltpu.async_copy(ptab.at[src_v.at[g * K_GRP + b]],
                                 rows_v.at[h, b], gsem.at[h])

        def drain_gathers(h, g):
            for b in range(K_GRP):
                pltpu.make_async_copy(ptab.at[src_v.at[g * K_GRP + b]],
                                      rows_v.at[h, b], gsem.at[h]).wait()

        def scatters(h, g):
            for b in range(K_GRP):
                pltpu.async_copy(rows_v.at[h, b],
                                 acc.at[dst_v.at[g * K_GRP + b]],
                                 ssem.at[h], add=True)

        def drain_scatters(h, g):
            for b in range(K_GRP):
                pltpu.make_async_copy(rows_v.at[h, b],
                                      acc.at[dst_v.at[g * K_GRP + b]],
                                      ssem.at[h]).wait()

        gathers(0, 0)

        def body(t, carry):
            g = 2 * t
            gathers(1, g + 1)
            drain_gathers(0, g)
            scatters(0, g)
            drain_scatters(0, g)

            @pl.when(g + 2 < NGRP)
            def _():
                gathers(0, g + 2)
            drain_gathers(1, g + 1)
            scatters(1, g + 1)
            drain_scatters(1, g + 1)
            return carry

        lax.fori_loop(0, NGRP // 2, body, 0)
        plsc.subcore_barrier()
        pltpu.sync_copy(acc.at[pl.ds(sid * RPT, RPT)],
                        out_hbm.at[pl.ds(cid * ACC_ROWS + sid * RPT, RPT)])

    return seg


_segsum_h1 = _make_segsum(H1_DIM)
_segsum_h2 = _make_segsum(H2_DIM)


N_BLK = 16
BLK = ACC_ROWS // N_BLK


def _tc_project(h, w):
    dh = w.shape[1] // 2

    def body(h_ref, wa_ref, wb_ref, o0_ref, o1_ref):
        o0_ref[...] = jnp.dot(h_ref[...], wa_ref[...],
                              preferred_element_type=jnp.float32)
        o1_ref[...] = jnp.dot(h_ref[...], wb_ref[...],
                              preferred_element_type=jnp.float32)

    return pl.pallas_call(
        body,
        grid=(N_BLK,),
        in_specs=[
            pl.BlockSpec((BLK, F_IN), lambda i: (i, 0)),
            pl.BlockSpec((F_IN, dh), lambda i: (0, 0)),
            pl.BlockSpec((F_IN, dh), lambda i: (0, 0)),
        ],
        out_specs=[
            pl.BlockSpec((BLK, dh), lambda i: (i, 0)),
            pl.BlockSpec((BLK, dh), lambda i: (i, 0)),
        ],
        out_shape=[
            jax.ShapeDtypeStruct((ACC_ROWS, dh), jnp.float32),
            jax.ShapeDtypeStruct((ACC_ROWS, dh), jnp.float32),
        ],
    )(h, w[:, :dh], w[:, dh:])


def _tc_mid(p, parts, m, b1, w1b, b1b, w2a):
    dh1 = H1_DIM // 2
    dh2 = w2a.shape[1] // 2
    p0, p1 = p
    pa0, pa1 = parts

    def body(p0_ref, p1_ref, pa0_ref, pa1_ref, m_ref, b1_ref, w1b_ref,
             b1b_ref, w2aa_ref, w2ab_ref, o0_ref, o1_ref):
        q0 = p0_ref[...] + pa0_ref[...]
        q1 = p1_ref[...] + pa1_ref[...]
        t = jnp.concatenate([q0, q1], axis=1) + b1_ref[...]
        t = jnp.maximum(t, 0.0)
        hh = jnp.dot(t, w1b_ref[...], preferred_element_type=jnp.float32)
        hh = (hh + b1b_ref[...]) * m_ref[...]
        o0_ref[...] = jnp.dot(hh, w2aa_ref[...],
                              preferred_element_type=jnp.float32)
        o1_ref[...] = jnp.dot(hh, w2ab_ref[...],
                              preferred_element_type=jnp.float32)

    return pl.pallas_call(
        body,
        grid=(N_BLK,),
        in_specs=[
            pl.BlockSpec((BLK, dh1), lambda i: (i, 0)),
            pl.BlockSpec((BLK, dh1), lambda i: (i, 0)),
            pl.BlockSpec((BLK, dh1), lambda i: (i, 0)),
            pl.BlockSpec((BLK, dh1), lambda i: (i, 0)),
            pl.BlockSpec((BLK, 1), lambda i: (i, 0)),
            pl.BlockSpec((1, H1_DIM), lambda i: (0, 0)),
            pl.BlockSpec((H1_DIM, H1_DIM), lambda i: (0, 0)),
            pl.BlockSpec((1, H1_DIM), lambda i: (0, 0)),
            pl.BlockSpec((H1_DIM, dh2), lambda i: (0, 0)),
            pl.BlockSpec((H1_DIM, dh2), lambda i: (0, 0)),
        ],
        out_specs=[
            pl.BlockSpec((BLK, dh2), lambda i: (i, 0)),
            pl.BlockSpec((BLK, dh2), lambda i: (i, 0)),
        ],
        out_shape=[
            jax.ShapeDtypeStruct((ACC_ROWS, dh2), jnp.float32),
            jax.ShapeDtypeStruct((ACC_ROWS, dh2), jnp.float32),
        ],
    )(p0, p1, pa0, pa1, m, b1, w1b, b1b, w2a[:, :dh2], w2a[:, dh2:])


def _tc_final(p, parts, m, b2, w2b, b2b, wf, bf):
    dh = H2_DIM // 2
    p0, p1 = p
    pa0, pa1 = parts

    def body(p0_ref, p1_ref, pa0_ref, pa1_ref, m_ref, b2_ref, w2b_ref,
             b2b_ref, wf_ref, bf_ref, o_ref):
        q0 = p0_ref[...] + pa0_ref[...]
        q1 = p1_ref[...] + pa1_ref[...]
        t = jnp.concatenate([q0, q1], axis=1) + b2_ref[...]
        t = jnp.maximum(t, 0.0)
        h = jnp.dot(t, w2b_ref[...], preferred_element_type=jnp.float32)
        h = (h + b2b_ref[...]) * m_ref[...]
        pooled = jnp.max(h, axis=0, keepdims=True)
        o_ref[...] = (jnp.dot(pooled, wf_ref[...],
                              preferred_element_type=jnp.float32)
                      + bf_ref[...])[None]

    return pl.pallas_call(
        body,
        grid=(B_G,),
        in_specs=[
            pl.BlockSpec((N_G, dh), lambda i: (i, 0)),
            pl.BlockSpec((N_G, dh), lambda i: (i, 0)),
            pl.BlockSpec((N_G, dh), lambda i: (i, 0)),
            pl.BlockSpec((N_G, dh), lambda i: (i, 0)),
            pl.BlockSpec((N_G, 1), lambda i: (i, 0)),
            pl.BlockSpec((1, H2_DIM), lambda i: (0, 0)),
            pl.BlockSpec((H2_DIM, H2_DIM), lambda i: (0, 0)),
            pl.BlockSpec((1, H2_DIM), lambda i: (0, 0)),
            pl.BlockSpec((H2_DIM, OUT_DIM), lambda i: (0, 0)),
            pl.BlockSpec((1, OUT_DIM), lambda i: (0, 0)),
        ],
        out_specs=pl.BlockSpec((1, 1, OUT_DIM), lambda i: (i, 0, 0)),
        out_shape=jax.ShapeDtypeStruct((B_G, 1, OUT_DIM), jnp.float32),
    )(p0, p1, pa0, pa1, m, b2, w2b, b2b, wf, bf).reshape(B_G, OUT_DIM)


def kernel(x, edge_index, mask, W1a, b1a, W1b, b1b, W2a, b2a, W2b, b2b, Wf, bf):
    h = x.reshape(NODES, F_IN)
    m = mask.reshape(NODES, 1)
    ei = edge_index.astype(jnp.int32)
    pad = E_PAD - E_EDGES
    src = jnp.concatenate([ei[0], jnp.zeros((pad,), jnp.int32)])
    dst = jnp.concatenate([ei[1], jnp.full((pad,), NODES, jnp.int32)])
    src = src.reshape(NS, NCHUNK, CHUNK)
    dst = dst.reshape(NS, NCHUNK, CHUNK)

    m_pad = jnp.pad(m, ((0, ACC_ROWS - NODES), (0, 0)))

    p1a, p1b = _tc_project(h, W1a)
    zeros1 = jnp.zeros((RPT, H1_DIM // 2), jnp.float32)
    parts1 = _segsum_h1(src, dst, p1a, p1b, zeros1)
    p2a, p2b = _tc_mid((p1a, p1b), (parts1[:ACC_ROWS], parts1[ACC_ROWS:]),
                       m_pad, b1a.reshape(1, H1_DIM), W1b,
                       b1b.reshape(1, H1_DIM), W2a)

    zeros2 = jnp.zeros((RPT, H2_DIM // 2), jnp.float32)
    parts2 = _segsum_h2(src, dst, p2a, p2b, zeros2)
    out = _tc_final((p2a, p2b), (parts2[:ACC_ROWS], parts2[ACC_ROWS:]),
                    m, b2a.reshape(1, H2_DIM), W2b,
                    b2b.reshape(1, H2_DIM), Wf, bf.reshape(1, OUT_DIM))
    return out


# speedup vs baseline: 1.1897x; 1.1897x over previous
import functools

import jax
import jax.numpy as jnp
from jax import lax
from jax.experimental import pallas as pl
from jax.experimental.pallas import tpu as pltpu
from jax.experimental.pallas import tpu_sc as plsc

B_G, N_G, F_IN = 10, 1000, 128
E_EDGES = 320000
H1_DIM, H2_DIM, OUT_DIM = 64, 32, 16
NODES = B_G * N_G

NC, NS = 2, 16
CHUNK = 128
NCHUNK = 160
E_PAD = NS * NCHUNK * CHUNK
ACC_ROWS = 10240
RPT = ACC_ROWS // NS

K_GRP = 4
NGRP = NCHUNK // K_GRP
E_ROWS = E_EDGES // CHUNK
PAD_ROWS = NS * NCHUNK


def _make_segsum(d: int):
    dh = d // 2
    mesh = plsc.VectorSubcoreMesh(core_axis_name="c", subcore_axis_name="s")

    @functools.partial(
        pl.kernel,
        out_type=pltpu.HBM((NC * ACC_ROWS, dh), jnp.float32),
        mesh=mesh,
        compiler_params=pltpu.CompilerParams(use_tc_tiling_on_sc=False),
        scratch_types=[
            pltpu.VMEM((NCHUNK, CHUNK), jnp.int32),
            pltpu.VMEM((NCHUNK, CHUNK), jnp.int32),
            pltpu.VMEM((2, K_GRP, CHUNK, dh), jnp.float32),
            pltpu.VMEM_SHARED((ACC_ROWS, dh), jnp.float32),
            pltpu.VMEM_SHARED((ACC_ROWS, dh), jnp.float32),
            pltpu.SemaphoreType.DMA((2,)),
            pltpu.SemaphoreType.DMA((2,)),
        ],
    )
    def seg(src_hbm, dst_hbm, p_hbm, zero_hbm, out_hbm,
            src_v, dst_v, rows_v, acc, ptab, gsem, ssem):
        cid = lax.axis_index("c")
        sid = lax.axis_index("s")
        pltpu.sync_copy(src_hbm.at[pl.ds(sid * NCHUNK, NCHUNK)], src_v)
        pltpu.sync_copy(dst_hbm.at[pl.ds(sid * NCHUNK, NCHUNK)], dst_v)
        pltpu.sync_copy(p_hbm.at[pl.ds(cid * ACC_ROWS + sid * RPT, RPT)],
                        ptab.at[pl.ds(sid * RPT, RPT)])
        pltpu.sync_copy(zero_hbm, acc.at[pl.ds(sid * RPT, RPT)])
        plsc.subcore_barrier()

        def gathers(h, g):
            for b in range(K_GRP):
                pltpu.async_copy(ptab.at[src_v.at[g * K_GRP + b]],
                                 rows_v.at[h, b], gsem.at[h])

        def drain_gathers(h, g):
            for b in range(K_GRP):
                pltpu.make_async_copy(ptab.at[src_v.at[g * K_GRP + b]],
                                      rows_v.at[h, b], gsem.at[h]).wait()

        def scatters(h, g):
            for b in range(K_GRP):
                pltpu.async_copy(rows_v.at[h, b],
                                 acc.at[dst_v.at[g * K_GRP + b]],
                                 ssem.at[h], add=True)

        def drain_scatters(h, g):
            for b in range(K_GRP):
                pltpu.make_async_copy(rows_v.at[h, b],
                                      acc.at[dst_v.at[g * K_GRP + b]],
                                      ssem.at[h]).wait()

        gathers(0, 0)

        def body(t, carry):
            g = 2 * t
            gathers(1, g + 1)
            drain_gathers(0, g)
            scatters(0, g)
            drain_scatters(0, g)

            @pl.when(g + 2 < NGRP)
            def _():
                gathers(0, g + 2)
            drain_gathers(1, g + 1)
            scatters(1, g + 1)
            drain_scatters(1, g + 1)
            return carry

        lax.fori_loop(0, NGRP // 2, body, 0)
        plsc.subcore_barrier()
        pltpu.sync_copy(acc.at[pl.ds(sid * RPT, RPT)],
                        out_hbm.at[pl.ds(cid * ACC_ROWS + sid * RPT, RPT)])

    return seg


_segsum_h1 = _make_segsum(H1_DIM)
_segsum_h2 = _make_segsum(H2_DIM)


def _tc_project(h, w, ei):
    dh = w.shape[1] // 2

    def body(h_ref, wa_ref, wb_ref, ei_ref, o_ref, src_ref, dst_ref):
        o_ref[0:NODES, :] = jnp.dot(h_ref[...], wa_ref[...],
                                    preferred_element_type=jnp.float32)
        o_ref[pl.ds(ACC_ROWS, NODES), :] = jnp.dot(
            h_ref[...], wb_ref[...], preferred_element_type=jnp.float32)
        tail = PAD_ROWS - (E_ROWS - E_ROWS % 8)
        src_ref[pl.ds(PAD_ROWS - tail, tail), :] = jnp.zeros(
            (tail, CHUNK), jnp.int32)
        dst_ref[pl.ds(PAD_ROWS - tail, tail), :] = jnp.full(
            (tail, CHUNK), NODES, jnp.int32)
        src_ref[0:E_ROWS, :] = ei_ref[0, :].reshape(E_ROWS, CHUNK)
        dst_ref[0:E_ROWS, :] = ei_ref[1, :].reshape(E_ROWS, CHUNK)

    return pl.pallas_call(
        body,
        out_shape=[
            jax.ShapeDtypeStruct((NC * ACC_ROWS, dh), jnp.float32),
            jax.ShapeDtypeStruct((PAD_ROWS, CHUNK), jnp.int32),
            jax.ShapeDtypeStruct((PAD_ROWS, CHUNK), jnp.int32),
        ],
    )(h, w[:, :dh], w[:, dh:], ei)


def _tc_mid(p, parts, m, b1, w1b, b1b, w2a):
    dh2 = w2a.shape[1] // 2

    def body(p_ref, pa_ref, m_ref, b1_ref, w1b_ref, b1b_ref,
             w2aa_ref, w2ab_ref, o_ref):
        q0 = p_ref[0:NODES, :] + pa_ref[0:NODES, :]
        q1 = p_ref[pl.ds(ACC_ROWS, NODES), :] + pa_ref[pl.ds(ACC_ROWS, NODES), :]
        t = jnp.concatenate([q0, q1], axis=1) + b1_ref[...]
        t = jnp.maximum(t, 0.0)
        hh = jnp.dot(t, w1b_ref[...], preferred_element_type=jnp.float32)
        hh = (hh + b1b_ref[...]) * m_ref[...]
        o_ref[0:NODES, :] = jnp.dot(hh, w2aa_ref[...],
                                    preferred_element_type=jnp.float32)
        o_ref[pl.ds(ACC_ROWS, NODES), :] = jnp.dot(
            hh, w2ab_ref[...], preferred_element_type=jnp.float32)

    return pl.pallas_call(
        body,
        out_shape=jax.ShapeDtypeStruct((NC * ACC_ROWS, dh2), jnp.float32),
    )(p, parts, m, b1, w1b, b1b, w2a[:, :dh2], w2a[:, dh2:])


def _tc_final(p, parts, m, b2, w2b, b2b, wf, bf):
    dh = H2_DIM // 2

    def body(p_ref, pa_ref, m_ref, b2_ref, w2b_ref, b2b_ref, wf_ref, bf_ref,
             o_ref):
        q0 = p_ref[0] + pa_ref[0]
        q1 = p_ref[1] + pa_ref[1]
        t = jnp.concatenate([q0, q1], axis=1) + b2_ref[...]
        t = jnp.maximum(t, 0.0)
        h = jnp.dot(t, w2b_ref[...], preferred_element_type=jnp.float32)
        h = (h + b2b_ref[...]) * m_ref[...]
        pooled = jnp.max(h, axis=0, keepdims=True)
        o_ref[...] = (jnp.dot(pooled, wf_ref[...],
                              preferred_element_type=jnp.float32)
                      + bf_ref[...])[None]

    return pl.pallas_call(
        body,
        grid=(B_G,),
        in_specs=[
            pl.BlockSpec((NC, N_G, dh), lambda i: (0, i, 0)),
            pl.BlockSpec((NC, N_G, dh), lambda i: (0, i, 0)),
            pl.BlockSpec((N_G, 1), lambda i: (i, 0)),
            pl.BlockSpec((1, H2_DIM), lambda i: (0, 0)),
            pl.BlockSpec((H2_DIM, H2_DIM), lambda i: (0, 0)),
            pl.BlockSpec((1, H2_DIM), lambda i: (0, 0)),
            pl.BlockSpec((H2_DIM, OUT_DIM), lambda i: (0, 0)),
            pl.BlockSpec((1, OUT_DIM), lambda i: (0, 0)),
        ],
        out_specs=pl.BlockSpec((1, 1, OUT_DIM), lambda i: (i, 0, 0)),
        out_shape=jax.ShapeDtypeStruct((B_G, 1, OUT_DIM), jnp.float32),
    )(p, parts, m, b2, w2b, b2b, wf, bf).reshape(B_G, OUT_DIM)


def kernel(x, edge_index, mask, W1a, b1a, W1b, b1b, W2a, b2a, W2b, b2b, Wf, bf):
    h = x.reshape(NODES, F_IN)
    m = mask.reshape(NODES, 1)
    ei = edge_index.astype(jnp.int32)

    p1, src, dst = _tc_project(h, W1a, ei)
    zeros1 = jnp.zeros((RPT, H1_DIM // 2), jnp.float32)
    parts1 = _segsum_h1(src, dst, p1, zeros1)
    p2 = _tc_mid(p1, parts1, m, b1a.reshape(1, H1_DIM), W1b,
                 b1b.reshape(1, H1_DIM), W2a)

    zeros2 = jnp.zeros((RPT, H2_DIM // 2), jnp.float32)
    parts2 = _segsum_h2(src, dst, p2, zeros2)
    out = _tc_final(p2.reshape(NC, ACC_ROWS, H2_DIM // 2),
                    parts2.reshape(NC, ACC_ROWS, H2_DIM // 2),
                    m, b2a.reshape(1, H2_DIM), W2b,
                    b2b.reshape(1, H2_DIM), Wf, bf.reshape(1, OUT_DIM))
    return out


# speedup vs baseline: 1.2452x; 1.0466x over previous
import functools

import jax
import jax.numpy as jnp
from jax import lax
from jax.experimental import pallas as pl
from jax.experimental.pallas import tpu as pltpu
from jax.experimental.pallas import tpu_sc as plsc

B_G, N_G, F_IN = 10, 1000, 128
E_EDGES = 320000
H1_DIM, H2_DIM, OUT_DIM = 64, 32, 16
NODES = B_G * N_G

NC, NS = 2, 16
CHUNK = 128
NCHUNK = 160
E_PAD = NS * NCHUNK * CHUNK
ACC_ROWS = 10240
RPT = ACC_ROWS // NS

K_GRP = 4
NGRP = NCHUNK // K_GRP
E_ROWS = E_EDGES // CHUNK
PAD_ROWS = NS * NCHUNK


def _make_segsum(d: int):
    dh = d // 2
    mesh = plsc.VectorSubcoreMesh(core_axis_name="c", subcore_axis_name="s")

    @functools.partial(
        pl.kernel,
        out_type=pltpu.HBM((NC * ACC_ROWS, dh), jnp.float32),
        mesh=mesh,
        compiler_params=pltpu.CompilerParams(use_tc_tiling_on_sc=False),
        scratch_types=[
            pltpu.VMEM((NCHUNK, CHUNK), jnp.int32),
            pltpu.VMEM((NCHUNK, CHUNK), jnp.int32),
            pltpu.VMEM((2, K_GRP, CHUNK, dh), jnp.float32),
            pltpu.VMEM_SHARED((ACC_ROWS, dh), jnp.float32),
            pltpu.VMEM_SHARED((ACC_ROWS, dh), jnp.float32),
            pltpu.SemaphoreType.DMA((2,)),
            pltpu.SemaphoreType.DMA((2,)),
        ],
    )
    def seg(src_hbm, dst_hbm, p_hbm, zero_hbm, out_hbm,
            src_v, dst_v, rows_v, acc, ptab, gsem, ssem):
        cid = lax.axis_index("c")
        sid = lax.axis_index("s")
        pltpu.sync_copy(src_hbm.at[pl.ds(sid * NCHUNK, NCHUNK)], src_v)
        pltpu.sync_copy(dst_hbm.at[pl.ds(sid * NCHUNK, NCHUNK)], dst_v)
        pltpu.sync_copy(p_hbm.at[pl.ds(cid * ACC_ROWS + sid * RPT, RPT)],
                        ptab.at[pl.ds(sid * RPT, RPT)])
        pltpu.sync_copy(zero_hbm, acc.at[pl.ds(sid * RPT, RPT)])
        plsc.subcore_barrier()

        def gathers(h, g):
            for b in range(K_GRP):
                pltpu.async_copy(ptab.at[src_v.at[g * K_GRP + b]],
                                 rows_v.at[h, b], gsem.at[h])

        def drain_gathers(h, g):
            for b in range(K_GRP):
                pltpu.make_async_copy(ptab.at[src_v.at[g * K_GRP + b]],
                                      rows_v.at[h, b], gsem.at[h]).wait()

        def scatters(h, g):
            for b in range(K_GRP):
                pltpu.async_copy(rows_v.at[h, b],
                                 acc.at[dst_v.at[g * K_GRP + b]],
                                 ssem.at[h], add=True)

        def drain_scatters(h, g):
            for b in range(K_GRP):
                pltpu.make_async_copy(rows_v.at[h, b],
                                      acc.at[dst_v.at[g * K_GRP + b]],
                                      ssem.at[h]).wait()

        gathers(0, 0)

        def body(t, carry):
            g = 2 * t
            gathers(1, g + 1)
            drain_gathers(0, g)
            scatters(0, g)
            drain_scatters(0, g)

            @pl.when(g + 2 < NGRP)
            def _():
                gathers(0, g + 2)
            drain_gathers(1, g + 1)
            scatters(1, g + 1)
            drain_scatters(1, g + 1)
            return carry

        lax.fori_loop(0, NGRP // 2, body, 0)
        plsc.subcore_barrier()
        pltpu.sync_copy(acc.at[pl.ds(sid * RPT, RPT)],
                        out_hbm.at[pl.ds(cid * ACC_ROWS + sid * RPT, RPT)])

    return seg


_segsum_h1 = _make_segsum(H1_DIM)

E_HALF_ROWS = PAD_ROWS // NC
NCHUNK2 = E_HALF_ROWS // NS
NGRP2 = NCHUNK2 // K_GRP


def _make_segsum_edges(d: int):
    mesh = plsc.VectorSubcoreMesh(core_axis_name="c", subcore_axis_name="s")

    @functools.partial(
        pl.kernel,
        out_type=[pltpu.HBM((ACC_ROWS, d), jnp.float32),
                  pltpu.HBM((ACC_ROWS, d), jnp.float32)],
        mesh=mesh,
        compiler_params=pltpu.CompilerParams(use_tc_tiling_on_sc=False),
        scratch_types=[
            pltpu.VMEM((NCHUNK2, CHUNK), jnp.int32),
            pltpu.VMEM((NCHUNK2, CHUNK), jnp.int32),
            pltpu.VMEM((2, K_GRP, CHUNK, d), jnp.float32),
            pltpu.VMEM_SHARED((ACC_ROWS, d), jnp.float32),
            pltpu.VMEM_SHARED((ACC_ROWS, d), jnp.float32),
            pltpu.SemaphoreType.DMA((2,)),
            pltpu.SemaphoreType.DMA((2,)),
        ],
    )
    def seg(src_hbm, dst_hbm, p_hbm, zero_hbm, out0_hbm, out1_hbm,
            src_v, dst_v, rows_v, acc, ptab, gsem, ssem):
        cid = lax.axis_index("c")
        sid = lax.axis_index("s")
        base = cid * E_HALF_ROWS + sid * NCHUNK2
        pltpu.sync_copy(src_hbm.at[pl.ds(base, NCHUNK2)], src_v)
        pltpu.sync_copy(dst_hbm.at[pl.ds(base, NCHUNK2)], dst_v)
        pltpu.sync_copy(p_hbm.at[pl.ds(sid * RPT, RPT)],
                        ptab.at[pl.ds(sid * RPT, RPT)])
        pltpu.sync_copy(zero_hbm, acc.at[pl.ds(sid * RPT, RPT)])
        plsc.subcore_barrier()

        def gathers(h, g):
            for b in range(K_GRP):
                pltpu.async_copy(ptab.at[src_v.at[g * K_GRP + b]],
                                 rows_v.at[h, b], gsem.at[h])

        def drain_gathers(h, g):
            for b in range(K_GRP):
                pltpu.make_async_copy(ptab.at[src_v.at[g * K_GRP + b]],
                                      rows_v.at[h, b], gsem.at[h]).wait()

        def scatters(h, g):
            for b in range(K_GRP):
                pltpu.async_copy(rows_v.at[h, b],
                                 acc.at[dst_v.at[g * K_GRP + b]],
                                 ssem.at[h], add=True)

        def drain_scatters(h, g):
            for b in range(K_GRP):
                pltpu.make_async_copy(rows_v.at[h, b],
                                      acc.at[dst_v.at[g * K_GRP + b]],
                                      ssem.at[h]).wait()

        gathers(0, 0)

        def body(t, carry):
            g = 2 * t
            gathers(1, g + 1)
            drain_gathers(0, g)
            scatters(0, g)
            drain_scatters(0, g)

            @pl.when(g + 2 < NGRP2)
            def _():
                gathers(0, g + 2)
            drain_gathers(1, g + 1)
            scatters(1, g + 1)
            drain_scatters(1, g + 1)
            return carry

        lax.fori_loop(0, NGRP2 // 2, body, 0)
        plsc.subcore_barrier()

        @pl.when(cid == 0)
        def _():
            pltpu.sync_copy(acc.at[pl.ds(sid * RPT, RPT)],
                            out0_hbm.at[pl.ds(sid * RPT, RPT)])

        @pl.when(cid == 1)
        def _():
            pltpu.sync_copy(acc.at[pl.ds(sid * RPT, RPT)],
                            out1_hbm.at[pl.ds(sid * RPT, RPT)])

    return seg


_segsum_h2 = _make_segsum_edges(H2_DIM)


def _tc_project(h, w, ei):
    dh = w.shape[1] // 2

    def body(h_ref, wa_ref, wb_ref, ei_ref, o_ref, src_ref, dst_ref):
        o_ref[0:NODES, :] = jnp.dot(h_ref[...], wa_ref[...],
                                    preferred_element_type=jnp.float32)
        o_ref[pl.ds(ACC_ROWS, NODES), :] = jnp.dot(
            h_ref[...], wb_ref[...], preferred_element_type=jnp.float32)
        tail = PAD_ROWS - (E_ROWS - E_ROWS % 8)
        src_ref[pl.ds(PAD_ROWS - tail, tail), :] = jnp.zeros(
            (tail, CHUNK), jnp.int32)
        dst_ref[pl.ds(PAD_ROWS - tail, tail), :] = jnp.full(
            (tail, CHUNK), NODES, jnp.int32)
        src_ref[0:E_ROWS, :] = ei_ref[0, :].reshape(E_ROWS, CHUNK)
        dst_ref[0:E_ROWS, :] = ei_ref[1, :].reshape(E_ROWS, CHUNK)

    return pl.pallas_call(
        body,
        out_shape=[
            jax.ShapeDtypeStruct((NC * ACC_ROWS, dh), jnp.float32),
            jax.ShapeDtypeStruct((PAD_ROWS, CHUNK), jnp.int32),
            jax.ShapeDtypeStruct((PAD_ROWS, CHUNK), jnp.int32),
        ],
    )(h, w[:, :dh], w[:, dh:], ei)


def _tc_mid(p, parts, m, b1, w1b, b1b, w2a):
    def body(p_ref, pa_ref, m_ref, b1_ref, w1b_ref, b1b_ref, w2a_ref, o_ref):
        q0 = p_ref[0:NODES, :] + pa_ref[0:NODES, :]
        q1 = p_ref[pl.ds(ACC_ROWS, NODES), :] + pa_ref[pl.ds(ACC_ROWS, NODES), :]
        t = jnp.concatenate([q0, q1], axis=1) + b1_ref[...]
        t = jnp.maximum(t, 0.0)
        hh = jnp.dot(t, w1b_ref[...], preferred_element_type=jnp.float32)
        hh = (hh + b1b_ref[...]) * m_ref[...]
        o_ref[0:NODES, :] = jnp.dot(hh, w2a_ref[...],
                                    preferred_element_type=jnp.float32)

    return pl.pallas_call(
        body,
        out_shape=jax.ShapeDtypeStruct((ACC_ROWS, w2a.shape[1]), jnp.float32),
    )(p, parts, m, b1, w1b, b1b, w2a)


def _tc_final(p, pa0, pa1, m, b2, w2b, b2b, wf, bf):
    def body(p_ref, pa0_ref, pa1_ref, m_ref, b2_ref, w2b_ref, b2b_ref,
             wf_ref, bf_ref, o_ref):
        t = p_ref[...] + pa0_ref[...] + pa1_ref[...] + b2_ref[...]
        t = jnp.maximum(t, 0.0)
        h = jnp.dot(t, w2b_ref[...], preferred_element_type=jnp.float32)
        h = (h + b2b_ref[...]) * m_ref[...]
        pooled = jnp.max(h, axis=0, keepdims=True)
        o_ref[...] = (jnp.dot(pooled, wf_ref[...],
                              preferred_element_type=jnp.float32)
                      + bf_ref[...])[None]

    return pl.pallas_call(
        body,
        grid=(B_G,),
        in_specs=[
            pl.BlockSpec((N_G, H2_DIM), lambda i: (i, 0)),
            pl.BlockSpec((N_G, H2_DIM), lambda i: (i, 0)),
            pl.BlockSpec((N_G, H2_DIM), lambda i: (i, 0)),
            pl.BlockSpec((N_G, 1), lambda i: (i, 0)),
            pl.BlockSpec((1, H2_DIM), lambda i: (0, 0)),
            pl.BlockSpec((H2_DIM, H2_DIM), lambda i: (0, 0)),
            pl.BlockSpec((1, H2_DIM), lambda i: (0, 0)),
            pl.BlockSpec((H2_DIM, OUT_DIM), lambda i: (0, 0)),
            pl.BlockSpec((1, OUT_DIM), lambda i: (0, 0)),
        ],
        out_specs=pl.BlockSpec((1, 1, OUT_DIM), lambda i: (i, 0, 0)),
        out_shape=jax.ShapeDtypeStruct((B_G, 1, OUT_DIM), jnp.float32),
    )(p, pa0, pa1, m, b2, w2b, b2b, wf, bf).reshape(B_G, OUT_DIM)


def kernel(x, edge_index, mask, W1a, b1a, W1b, b1b, W2a, b2a, W2b, b2b, Wf, bf):
    h = x.reshape(NODES, F_IN)
    m = mask.reshape(NODES, 1)
    ei = edge_index.astype(jnp.int32)

    p1, src, dst = _tc_project(h, W1a, ei)
    zeros1 = jnp.zeros((RPT, H1_DIM // 2), jnp.float32)
    parts1 = _segsum_h1(src, dst, p1, zeros1)
    p2 = _tc_mid(p1, parts1, m, b1a.reshape(1, H1_DIM), W1b,
                 b1b.reshape(1, H1_DIM), W2a)

    zeros2 = jnp.zeros((RPT, H2_DIM), jnp.float32)
    parts2a, parts2b = _segsum_h2(src, dst, p2, zeros2)
    out = _tc_final(p2, parts2a, parts2b,
                    m, b2a.reshape(1, H2_DIM), W2b,
                    b2b.reshape(1, H2_DIM), Wf, bf.reshape(1, OUT_DIM))
    return out


# speedup vs baseline: 1.2520x; 1.0054x over previous
import functools

import jax
import jax.numpy as jnp
from jax import lax
from jax.experimental import pallas as pl
from jax.experimental.pallas import tpu as pltpu
from jax.experimental.pallas import tpu_sc as plsc

B_G, N_G, F_IN = 10, 1000, 128
E_EDGES = 320000
H1_DIM, H2_DIM, OUT_DIM = 64, 32, 16
NODES = B_G * N_G

NC, NS = 2, 16
CHUNK = 128
NCHUNK = 160
E_PAD = NS * NCHUNK * CHUNK
ACC_ROWS = 10240
RPT = ACC_ROWS // NS

K_GRP = 5
NGRP = NCHUNK // K_GRP
E_ROWS = E_EDGES // CHUNK
PAD_ROWS = NS * NCHUNK


def _make_segsum(d: int):
    dh = d // 2
    mesh = plsc.VectorSubcoreMesh(core_axis_name="c", subcore_axis_name="s")

    @functools.partial(
        pl.kernel,
        out_type=pltpu.HBM((NC * ACC_ROWS, dh), jnp.float32),
        mesh=mesh,
        compiler_params=pltpu.CompilerParams(use_tc_tiling_on_sc=False),
        scratch_types=[
            pltpu.VMEM((NCHUNK, CHUNK), jnp.int32),
            pltpu.VMEM((NCHUNK, CHUNK), jnp.int32),
            pltpu.VMEM((2, K_GRP, CHUNK, dh), jnp.float32),
            pltpu.VMEM_SHARED((ACC_ROWS, dh), jnp.float32),
            pltpu.VMEM_SHARED((ACC_ROWS, dh), jnp.float32),
            pltpu.SemaphoreType.DMA((2,)),
            pltpu.SemaphoreType.DMA((2,)),
        ],
    )
    def seg(src_hbm, dst_hbm, p_hbm, zero_hbm, out_hbm,
            src_v, dst_v, rows_v, acc, ptab, gsem, ssem):
        cid = lax.axis_index("c")
        sid = lax.axis_index("s")
        pltpu.sync_copy(src_hbm.at[pl.ds(sid * NCHUNK, NCHUNK)], src_v)
        pltpu.sync_copy(dst_hbm.at[pl.ds(sid * NCHUNK, NCHUNK)], dst_v)
        pltpu.sync_copy(p_hbm.at[pl.ds(cid * ACC_ROWS + sid * RPT, RPT)],
                        ptab.at[pl.ds(sid * RPT, RPT)])
        pltpu.sync_copy(zero_hbm, acc.at[pl.ds(sid * RPT, RPT)])
        plsc.subcore_barrier()

        def gathers(h, g):
            for b in range(K_GRP):
                pltpu.async_copy(ptab.at[src_v.at[g * K_GRP + b]],
                                 rows_v.at[h, b], gsem.at[h])

        def drain_gathers(h, g):
            for b in range(K_GRP):
                pltpu.make_async_copy(ptab.at[src_v.at[g * K_GRP + b]],
                                      rows_v.at[h, b], gsem.at[h]).wait()

        def scatters(h, g):
            for b in range(K_GRP):
                pltpu.async_copy(rows_v.at[h, b],
                                 acc.at[dst_v.at[g * K_GRP + b]],
                                 ssem.at[h], add=True)

        def drain_scatters(h, g):
            for b in range(K_GRP):
                pltpu.make_async_copy(rows_v.at[h, b],
                                      acc.at[dst_v.at[g * K_GRP + b]],
                                      ssem.at[h]).wait()

        gathers(0, 0)

        def body(t, carry):
            g = 2 * t
            gathers(1, g + 1)
            drain_gathers(0, g)
            scatters(0, g)
            drain_scatters(0, g)

            @pl.when(g + 2 < NGRP)
            def _():
                gathers(0, g + 2)
            drain_gathers(1, g + 1)
            scatters(1, g + 1)
            drain_scatters(1, g + 1)
            return carry

        lax.fori_loop(0, NGRP // 2, body, 0)
        plsc.subcore_barrier()
        pltpu.sync_copy(acc.at[pl.ds(sid * RPT, RPT)],
                        out_hbm.at[pl.ds(cid * ACC_ROWS + sid * RPT, RPT)])

    return seg


_segsum_h1 = _make_segsum(H1_DIM)

E_HALF_ROWS = PAD_ROWS // NC
NCHUNK2 = E_HALF_ROWS // NS
NGRP2 = NCHUNK2 // K_GRP


def _make_segsum_edges(d: int):
    mesh = plsc.VectorSubcoreMesh(core_axis_name="c", subcore_axis_name="s")

    @functools.partial(
        pl.kernel,
        out_type=[pltpu.HBM((ACC_ROWS, d), jnp.float32),
                  pltpu.HBM((ACC_ROWS, d), jnp.float32)],
        mesh=mesh,
        compiler_params=pltpu.CompilerParams(use_tc_tiling_on_sc=False),
        scratch_types=[
            pltpu.VMEM((NCHUNK2, CHUNK), jnp.int32),
            pltpu.VMEM((NCHUNK2, CHUNK), jnp.int32),
            pltpu.VMEM((2, K_GRP, CHUNK, d), jnp.float32),
            pltpu.VMEM_SHARED((ACC_ROWS, d), jnp.float32),
            pltpu.VMEM_SHARED((ACC_ROWS, d), jnp.float32),
            pltpu.SemaphoreType.DMA((2,)),
            pltpu.SemaphoreType.DMA((2,)),
        ],
    )
    def seg(src_hbm, dst_hbm, p_hbm, zero_hbm, out0_hbm, out1_hbm,
            src_v, dst_v, rows_v, acc, ptab, gsem, ssem):
        cid = lax.axis_index("c")
        sid = lax.axis_index("s")
        base = cid * E_HALF_ROWS + sid * NCHUNK2
        pltpu.sync_copy(src_hbm.at[pl.ds(base, NCHUNK2)], src_v)
        pltpu.sync_copy(dst_hbm.at[pl.ds(base, NCHUNK2)], dst_v)
        pltpu.sync_copy(p_hbm.at[pl.ds(sid * RPT, RPT)],
                        ptab.at[pl.ds(sid * RPT, RPT)])
        pltpu.sync_copy(zero_hbm, acc.at[pl.ds(sid * RPT, RPT)])
        plsc.subcore_barrier()

        def gathers(h, g):
            for b in range(K_GRP):
                pltpu.async_copy(ptab.at[src_v.at[g * K_GRP + b]],
                                 rows_v.at[h, b], gsem.at[h])

        def drain_gathers(h, g):
            for b in range(K_GRP):
                pltpu.make_async_copy(ptab.at[src_v.at[g * K_GRP + b]],
                                      rows_v.at[h, b], gsem.at[h]).wait()

        def scatters(h, g):
            for b in range(K_GRP):
                pltpu.async_copy(rows_v.at[h, b],
                                 acc.at[dst_v.at[g * K_GRP + b]],
                                 ssem.at[h], add=True)

        def drain_scatters(h, g):
            for b in range(K_GRP):
                pltpu.make_async_copy(rows_v.at[h, b],
                                      acc.at[dst_v.at[g * K_GRP + b]],
                                      ssem.at[h]).wait()

        gathers(0, 0)

        def body(t, carry):
            g = 2 * t
            gathers(1, g + 1)
            drain_gathers(0, g)
            scatters(0, g)
            drain_scatters(0, g)

            @pl.when(g + 2 < NGRP2)
            def _():
                gathers(0, g + 2)
            drain_gathers(1, g + 1)
            scatters(1, g + 1)
            drain_scatters(1, g + 1)
            return carry

        lax.fori_loop(0, NGRP2 // 2, body, 0)
        plsc.subcore_barrier()

        @pl.when(cid == 0)
        def _():
            pltpu.sync_copy(acc.at[pl.ds(sid * RPT, RPT)],
                            out0_hbm.at[pl.ds(sid * RPT, RPT)])

        @pl.when(cid == 1)
        def _():
            pltpu.sync_copy(acc.at[pl.ds(sid * RPT, RPT)],
                            out1_hbm.at[pl.ds(sid * RPT, RPT)])

    return seg


_segsum_h2 = _make_segsum_edges(H2_DIM)


def _tc_project(h, w, ei):
    dh = w.shape[1] // 2

    def body(h_ref, wa_ref, wb_ref, ei_ref, o_ref, src_ref, dst_ref):
        o_ref[0:NODES, :] = jnp.dot(h_ref[...], wa_ref[...],
                                    preferred_element_type=jnp.float32)
        o_ref[pl.ds(ACC_ROWS, NODES), :] = jnp.dot(
            h_ref[...], wb_ref[...], preferred_element_type=jnp.float32)
        tail = PAD_ROWS - (E_ROWS - E_ROWS % 8)
        src_ref[pl.ds(PAD_ROWS - tail, tail), :] = jnp.zeros(
            (tail, CHUNK), jnp.int32)
        dst_ref[pl.ds(PAD_ROWS - tail, tail), :] = jnp.full(
            (tail, CHUNK), NODES, jnp.int32)
        src_ref[0:E_ROWS, :] = ei_ref[0, :].reshape(E_ROWS, CHUNK)
        dst_ref[0:E_ROWS, :] = ei_ref[1, :].reshape(E_ROWS, CHUNK)

    return pl.pallas_call(
        body,
        out_shape=[
            jax.ShapeDtypeStruct((NC * ACC_ROWS, dh), jnp.float32),
            jax.ShapeDtypeStruct((PAD_ROWS, CHUNK), jnp.int32),
            jax.ShapeDtypeStruct((PAD_ROWS, CHUNK), jnp.int32),
        ],
    )(h, w[:, :dh], w[:, dh:], ei)


def _tc_mid(p, parts, m, b1, w1b, b1b, w2a):
    def body(p_ref, pa_ref, m_ref, b1_ref, w1b_ref, b1b_ref, w2a_ref, o_ref):
        q0 = p_ref[0:NODES, :] + pa_ref[0:NODES, :]
        q1 = p_ref[pl.ds(ACC_ROWS, NODES), :] + pa_ref[pl.ds(ACC_ROWS, NODES), :]
        t = jnp.concatenate([q0, q1], axis=1) + b1_ref[...]
        t = jnp.maximum(t, 0.0)
        hh = jnp.dot(t, w1b_ref[...], preferred_element_type=jnp.float32)
        hh = (hh + b1b_ref[...]) * m_ref[...]
        o_ref[0:NODES, :] = jnp.dot(hh, w2a_ref[...],
                                    preferred_element_type=jnp.float32)

    return pl.pallas_call(
        body,
        out_shape=jax.ShapeDtypeStruct((ACC_ROWS, w2a.shape[1]), jnp.float32),
    )(p, parts, m, b1, w1b, b1b, w2a)


def _tc_final(p, pa0, pa1, m, b2, w2b, b2b, wf, bf):
    def body(p_ref, pa0_ref, pa1_ref, m_ref, b2_ref, w2b_ref, b2b_ref,
             wf_ref, bf_ref, o_ref):
        t = p_ref[...] + pa0_ref[...] + pa1_ref[...] + b2_ref[...]
        t = jnp.maximum(t, 0.0)
        h = jnp.dot(t, w2b_ref[...], preferred_element_type=jnp.float32)
        h = (h + b2b_ref[...]) * m_ref[...]
        pooled = jnp.max(h, axis=0, keepdims=True)
        o_ref[...] = (jnp.dot(pooled, wf_ref[...],
                              preferred_element_type=jnp.float32)
                      + bf_ref[...])[None]

    return pl.pallas_call(
        body,
        grid=(B_G,),
        in_specs=[
            pl.BlockSpec((N_G, H2_DIM), lambda i: (i, 0)),
            pl.BlockSpec((N_G, H2_DIM), lambda i: (i, 0)),
            pl.BlockSpec((N_G, H2_DIM), lambda i: (i, 0)),
            pl.BlockSpec((N_G, 1), lambda i: (i, 0)),
            pl.BlockSpec((1, H2_DIM), lambda i: (0, 0)),
            pl.BlockSpec((H2_DIM, H2_DIM), lambda i: (0, 0)),
            pl.BlockSpec((1, H2_DIM), lambda i: (0, 0)),
            pl.BlockSpec((H2_DIM, OUT_DIM), lambda i: (0, 0)),
            pl.BlockSpec((1, OUT_DIM), lambda i: (0, 0)),
        ],
        out_specs=pl.BlockSpec((1, 1, OUT_DIM), lambda i: (i, 0, 0)),
        out_shape=jax.ShapeDtypeStruct((B_G, 1, OUT_DIM), jnp.float32),
    )(p, pa0, pa1, m, b2, w2b, b2b, wf, bf).reshape(B_G, OUT_DIM)


def kernel(x, edge_index, mask, W1a, b1a, W1b, b1b, W2a, b2a, W2b, b2b, Wf, bf):
    h = x.reshape(NODES, F_IN)
    m = mask.reshape(NODES, 1)
    ei = edge_index.astype(jnp.int32)

    p1, src, dst = _tc_project(h, W1a, ei)
    zeros1 = jnp.zeros((RPT, H1_DIM // 2), jnp.float32)
    parts1 = _segsum_h1(src, dst, p1, zeros1)
    p2 = _tc_mid(p1, parts1, m, b1a.reshape(1, H1_DIM), W1b,
                 b1b.reshape(1, H1_DIM), W2a)

    zeros2 = jnp.zeros((RPT, H2_DIM), jnp.float32)
    parts2a, parts2b = _segsum_h2(src, dst, p2, zeros2)
    out = _tc_final(p2, parts2a, parts2b,
                    m, b2a.reshape(1, H2_DIM), W2b,
                    b2b.reshape(1, H2_DIM), Wf, bf.reshape(1, OUT_DIM))
    return out
